# Initial kernel scaffold; baseline (speedup 1.0000x reference)
import jax
import jax.numpy as jnp
from jax.experimental import pallas as pl


def kernel(data, edge_index, W1, b1, W2, b2):
    raise NotImplementedError("write your pallas kernel here")


# speedup vs baseline: 11.7571x; 11.7571x over previous
import functools

import jax
import jax.numpy as jnp
from jax import lax
from jax.experimental import pallas as pl
from jax.experimental.pallas import tpu as pltpu
from jax.experimental.pallas import tpu_sc as plsc

N = 10000
E = 320000
D = 128
HD = D // 2
NP = 10240

NC = 2
NS = 16
NW = NC * NS
K = 80
EPW = E // NW
NCH = EPW // K
EPT = E // NS
KCH = EPT // K
R = NP // NS
ZR = 128

_mesh = plsc.VectorSubcoreMesh(core_axis_name="c", subcore_axis_name="s")


@functools.partial(
    pl.kernel,
    out_type=jax.ShapeDtypeStruct((NC, NP, 16), jnp.float32),
    mesh=_mesh,
    scratch_types=[
        pltpu.VMEM((NCH, K), jnp.int32),
        pltpu.VMEM((K, 16), jnp.float32),
        pltpu.VMEM((R, 16), jnp.float32),
        pltpu.VMEM_SHARED((NP, 16), jnp.float32),
    ],
    compiler_params=pltpu.CompilerParams(use_tc_tiling_on_sc=False),
)
def _deg_sc(dst_hbm, out_hbm, idx_v, ones_v, zero_v, acc_sh):
    c = lax.axis_index("c")
    s = lax.axis_index("s")
    wid = c * NS + s
    pltpu.sync_copy(dst_hbm.at[wid], idx_v)

    def _fill(i, _):
        ones_v[i] = jnp.ones((16,), jnp.float32)
        return 0

    lax.fori_loop(0, K, _fill, 0)

    def _fillz(i, _):
        zero_v[i] = jnp.zeros((16,), jnp.float32)
        return 0

    lax.fori_loop(0, R, _fillz, 0)
    pltpu.sync_copy(zero_v, acc_sh.at[pl.ds(s * R, R)])
    plsc.subcore_barrier()

    def _body(j, _):
        pltpu.sync_copy(ones_v, acc_sh.at[idx_v.at[j]], add=True)
        return 0

    lax.fori_loop(0, NCH, _body, 0)
    plsc.subcore_barrier()
    pltpu.sync_copy(acc_sh.at[pl.ds(s * R, R)], out_hbm.at[c].at[pl.ds(s * R, R)])


@functools.partial(
    pl.kernel,
    out_type=jax.ShapeDtypeStruct((NC, NP, HD), jnp.float32),
    mesh=_mesh,
    scratch_types=[
        pltpu.VMEM((KCH, K), jnp.int32),
        pltpu.VMEM((KCH, K), jnp.int32),
        pltpu.VMEM((K, HD), jnp.float32),
        pltpu.VMEM((ZR, HD), jnp.float32),
        pltpu.VMEM_SHARED((NP, HD), jnp.float32),
        pltpu.SemaphoreType.DMA,
    ],
    compiler_params=pltpu.CompilerParams(use_tc_tiling_on_sc=False),
)
def _edge_sc(table_hbm, src_hbm, dst_hbm, out_hbm,
             src_v, dst_v, rows_v, zero_v, acc_sh, sem):
    c = lax.axis_index("c")
    s = lax.axis_index("s")
    pltpu.sync_copy(src_hbm.at[s], src_v)
    pltpu.sync_copy(dst_hbm.at[s], dst_v)

    def _fillz(i, _):
        for q in range(HD // 16):
            zero_v[i, pl.ds(q * 16, 16)] = jnp.zeros((16,), jnp.float32)
        return 0

    lax.fori_loop(0, ZR, _fillz, 0)
    for t in range(R // ZR):
        pltpu.sync_copy(zero_v, acc_sh.at[pl.ds(s * R + t * ZR, ZR)])
    plsc.subcore_barrier()

    def _body(j, _):
        pltpu.async_copy(table_hbm.at[c].at[src_v.at[j]], rows_v, sem).wait()
        pltpu.sync_copy(rows_v, acc_sh.at[dst_v.at[j]], add=True)
        return 0

    lax.fori_loop(0, KCH, _body, 0)
    plsc.subcore_barrier()
    pltpu.sync_copy(acc_sh.at[pl.ds(s * R, R)], out_hbm.at[c].at[pl.ds(s * R, R)])


_BR = 1024


def _norm_block(degp):
    deg = degp[0, :, 0:1] + degp[1, :, 0:1] + 1.0
    return lax.rsqrt(deg)


def _split_cols(m):
    return jnp.stack([m[:, :HD], m[:, HD:]])


def _tc_pre_body(x_ref, w_ref, degp_ref, table_ref, self_ref):
    norm = _norm_block(degp_ref[...])
    h = jnp.dot(x_ref[...], w_ref[...], preferred_element_type=jnp.float32)
    table_ref[...] = _split_cols(h * norm)
    self_ref[...] = h * (norm * norm)


def _tc_mid_body(p_ref, degp_ref, self_ref, b_ref, w_ref,
                 x_ref, table_ref, self2_ref):
    norm = _norm_block(degp_ref[...])
    p = jnp.concatenate([p_ref[0], p_ref[1]], axis=1)
    agg = p * norm + self_ref[...] + b_ref[...]
    x1 = jnp.maximum(agg, 0.0)
    x_ref[...] = x1
    h2 = jnp.dot(x1, w_ref[...], preferred_element_type=jnp.float32)
    table_ref[...] = _split_cols(h2 * norm)
    self2_ref[...] = h2 * (norm * norm)


def _tc_post_body(p_ref, degp_ref, self_ref, b_ref, x_ref):
    norm = _norm_block(degp_ref[...])
    p = jnp.concatenate([p_ref[0], p_ref[1]], axis=1)
    agg = p * norm + self_ref[...] + b_ref[...]
    x_ref[...] = jnp.maximum(agg, 0.0)


_row_spec = pl.BlockSpec((_BR, D), lambda i: (i, 0))
_t_spec = pl.BlockSpec((NC, _BR, HD), lambda i: (0, i, 0))
_degp_spec = pl.BlockSpec((NC, _BR, 16), lambda i: (0, i, 0))
_w_spec = pl.BlockSpec((D, D), lambda i: (0, 0))
_b_spec = pl.BlockSpec((1, D), lambda i: (0, 0))
_grid = (NP // _BR,)

_tc_pre = pl.pallas_call(
    _tc_pre_body,
    grid=_grid,
    in_specs=[_row_spec, _w_spec, _degp_spec],
    out_specs=[_t_spec, _row_spec],
    out_shape=[jax.ShapeDtypeStruct((NC, NP, HD), jnp.float32),
               jax.ShapeDtypeStruct((NP, D), jnp.float32)],
)

_tc_mid = pl.pallas_call(
    _tc_mid_body,
    grid=_grid,
    in_specs=[_t_spec, _degp_spec, _row_spec, _b_spec, _w_spec],
    out_specs=[_row_spec, _t_spec, _row_spec],
    out_shape=[jax.ShapeDtypeStruct((NP, D), jnp.float32),
               jax.ShapeDtypeStruct((NC, NP, HD), jnp.float32),
               jax.ShapeDtypeStruct((NP, D), jnp.float32)],
)

_tc_post = pl.pallas_call(
    _tc_post_body,
    grid=_grid,
    in_specs=[_t_spec, _degp_spec, _row_spec, _b_spec],
    out_specs=_row_spec,
    out_shape=jax.ShapeDtypeStruct((NP, D), jnp.float32),
)


def kernel(data, edge_index, W1, b1, W2, b2):
    srcd = edge_index[0].reshape(NW, NCH, K)
    dstd = edge_index[1].reshape(NW, NCH, K)
    srce = edge_index[0].reshape(NS, KCH, K)
    dste = edge_index[1].reshape(NS, KCH, K)
    data_p = jnp.pad(data, ((0, NP - N), (0, 0)))
    degp = _deg_sc(dstd)
    table1, self1 = _tc_pre(data_p, W1, degp)
    p1 = _edge_sc(table1, srce, dste)
    x1, table2, self2 = _tc_mid(p1, degp, self1, b1.reshape(1, D), W2)
    p2 = _edge_sc(table2, srce, dste)
    x2 = _tc_post(p2, degp, self2, b2.reshape(1, D))
    return (x2[:N], x1[:N], x2[:N])


# speedup vs baseline: 18.6508x; 1.5863x over previous
import functools

import jax
import jax.numpy as jnp
from jax import lax
from jax.experimental import pallas as pl
from jax.experimental.pallas import tpu as pltpu
from jax.experimental.pallas import tpu_sc as plsc

N = 10000
E = 320000
D = 128
HD = D // 2
NP = 10240

NC = 2
NS = 16
NW = NC * NS
K = 80
EPW = E // NW
NCH = EPW // K
EPT = E // NS
KCH = EPT // K
R = NP // NS
ZR = 128

_mesh = plsc.VectorSubcoreMesh(core_axis_name="c", subcore_axis_name="s")


@functools.partial(
    pl.kernel,
    out_type=jax.ShapeDtypeStruct((NC, NP, 16), jnp.float32),
    mesh=_mesh,
    scratch_types=[
        pltpu.VMEM((NCH, K), jnp.int32),
        pltpu.VMEM((K, 16), jnp.float32),
        pltpu.VMEM((R, 16), jnp.float32),
        pltpu.VMEM_SHARED((NP, 16), jnp.float32),
    ],
    compiler_params=pltpu.CompilerParams(use_tc_tiling_on_sc=False),
)
def _deg_sc(dst_hbm, out_hbm, idx_v, ones_v, zero_v, acc_sh):
    c = lax.axis_index("c")
    s = lax.axis_index("s")
    wid = c * NS + s
    pltpu.sync_copy(dst_hbm.at[wid], idx_v)

    def _fill(i, _):
        ones_v[i] = jnp.ones((16,), jnp.float32)
        return 0

    lax.fori_loop(0, K, _fill, 0)

    def _fillz(i, _):
        zero_v[i] = jnp.zeros((16,), jnp.float32)
        return 0

    lax.fori_loop(0, R, _fillz, 0)
    pltpu.sync_copy(zero_v, acc_sh.at[pl.ds(s * R, R)])
    plsc.subcore_barrier()

    def _body(j, _):
        pltpu.sync_copy(ones_v, acc_sh.at[idx_v.at[j]], add=True)
        return 0

    lax.fori_loop(0, NCH, _body, 0)
    plsc.subcore_barrier()
    pltpu.sync_copy(acc_sh.at[pl.ds(s * R, R)], out_hbm.at[c].at[pl.ds(s * R, R)])


@functools.partial(
    pl.kernel,
    out_type=jax.ShapeDtypeStruct((NC, NP, HD), jnp.float32),
    mesh=_mesh,
    scratch_types=[
        pltpu.VMEM((KCH, K), jnp.int32),
        pltpu.VMEM((KCH, K), jnp.int32),
        pltpu.VMEM((K, HD), jnp.float32),
        pltpu.VMEM((K, HD), jnp.float32),
        pltpu.VMEM((ZR, HD), jnp.float32),
        pltpu.VMEM_SHARED((NP, HD), jnp.float32),
        pltpu.SemaphoreType.DMA,
        pltpu.SemaphoreType.DMA,
        pltpu.SemaphoreType.DMA,
        pltpu.SemaphoreType.DMA,
    ],
    compiler_params=pltpu.CompilerParams(use_tc_tiling_on_sc=False),
)
def _edge_sc(table_hbm, src_hbm, dst_hbm, out_hbm,
             src_v, dst_v, rows0_v, rows1_v, zero_v, acc_sh,
             sem0, sem1, isem0, isem1):
    c = lax.axis_index("c")
    s = lax.axis_index("s")
    tb = table_hbm.at[c]
    pltpu.async_copy(src_hbm.at[s], src_v, isem0)
    pltpu.async_copy(dst_hbm.at[s], dst_v, isem1)

    def _fillz(i, _):
        for q in range(HD // 16):
            zero_v[i, pl.ds(q * 16, 16)] = jnp.zeros((16,), jnp.float32)
        return 0

    lax.fori_loop(0, ZR, _fillz, 0)
    for t in range(R // ZR):
        pltpu.sync_copy(zero_v, acc_sh.at[pl.ds(s * R + t * ZR, ZR)])
    pltpu.make_async_copy(src_hbm.at[s], src_v, isem0).wait()
    pltpu.make_async_copy(dst_hbm.at[s], dst_v, isem1).wait()
    plsc.subcore_barrier()

    pltpu.async_copy(tb.at[src_v.at[0]], rows0_v, sem0)
    pltpu.async_copy(tb.at[src_v.at[1]], rows1_v, sem1)

    def _body(jj, _):
        j0 = 2 * jj
        pltpu.make_async_copy(tb.at[src_v.at[j0]], rows0_v, sem0).wait()
        pltpu.sync_copy(rows0_v, acc_sh.at[dst_v.at[j0]], add=True)

        @pl.when(j0 + 2 < KCH)
        def _():
            pltpu.async_copy(tb.at[src_v.at[j0 + 2]], rows0_v, sem0)

        pltpu.make_async_copy(tb.at[src_v.at[j0 + 1]], rows1_v, sem1).wait()
        pltpu.sync_copy(rows1_v, acc_sh.at[dst_v.at[j0 + 1]], add=True)

        @pl.when(j0 + 3 < KCH)
        def _():
            pltpu.async_copy(tb.at[src_v.at[j0 + 3]], rows1_v, sem1)

        return 0

    lax.fori_loop(0, KCH // 2, _body, 0)
    plsc.subcore_barrier()
    pltpu.sync_copy(acc_sh.at[pl.ds(s * R, R)], out_hbm.at[c].at[pl.ds(s * R, R)])


_BR = 1024


def _norm_block(degp):
    deg = degp[0, :, 0:1] + degp[1, :, 0:1] + 1.0
    return lax.rsqrt(deg)


def _split_cols(m):
    return jnp.stack([m[:, :HD], m[:, HD:]])


def _tc_pre_body(x_ref, w_ref, degp_ref, table_ref, self_ref):
    norm = _norm_block(degp_ref[...])
    h = jnp.dot(x_ref[...], w_ref[...], preferred_element_type=jnp.float32)
    table_ref[...] = _split_cols(h * norm)
    self_ref[...] = h * (norm * norm)


def _tc_mid_body(p_ref, degp_ref, self_ref, b_ref, w_ref,
                 x_ref, table_ref, self2_ref):
    norm = _norm_block(degp_ref[...])
    p = jnp.concatenate([p_ref[0], p_ref[1]], axis=1)
    agg = p * norm + self_ref[...] + b_ref[...]
    x1 = jnp.maximum(agg, 0.0)
    x_ref[...] = x1
    h2 = jnp.dot(x1, w_ref[...], preferred_element_type=jnp.float32)
    table_ref[...] = _split_cols(h2 * norm)
    self2_ref[...] = h2 * (norm * norm)


def _tc_post_body(p_ref, degp_ref, self_ref, b_ref, x_ref):
    norm = _norm_block(degp_ref[...])
    p = jnp.concatenate([p_ref[0], p_ref[1]], axis=1)
    agg = p * norm + self_ref[...] + b_ref[...]
    x_ref[...] = jnp.maximum(agg, 0.0)


_row_spec = pl.BlockSpec((_BR, D), lambda i: (i, 0))
_t_spec = pl.BlockSpec((NC, _BR, HD), lambda i: (0, i, 0))
_degp_spec = pl.BlockSpec((NC, _BR, 16), lambda i: (0, i, 0))
_w_spec = pl.BlockSpec((D, D), lambda i: (0, 0))
_b_spec = pl.BlockSpec((1, D), lambda i: (0, 0))
_grid = (NP // _BR,)

_tc_pre = pl.pallas_call(
    _tc_pre_body,
    grid=_grid,
    in_specs=[_row_spec, _w_spec, _degp_spec],
    out_specs=[_t_spec, _row_spec],
    out_shape=[jax.ShapeDtypeStruct((NC, NP, HD), jnp.float32),
               jax.ShapeDtypeStruct((NP, D), jnp.float32)],
)

_tc_mid = pl.pallas_call(
    _tc_mid_body,
    grid=_grid,
    in_specs=[_t_spec, _degp_spec, _row_spec, _b_spec, _w_spec],
    out_specs=[_row_spec, _t_spec, _row_spec],
    out_shape=[jax.ShapeDtypeStruct((NP, D), jnp.float32),
               jax.ShapeDtypeStruct((NC, NP, HD), jnp.float32),
               jax.ShapeDtypeStruct((NP, D), jnp.float32)],
)

_tc_post = pl.pallas_call(
    _tc_post_body,
    grid=_grid,
    in_specs=[_t_spec, _degp_spec, _row_spec, _b_spec],
    out_specs=_row_spec,
    out_shape=jax.ShapeDtypeStruct((NP, D), jnp.float32),
)


def kernel(data, edge_index, W1, b1, W2, b2):
    srcd = edge_index[0].reshape(NW, NCH, K)
    dstd = edge_index[1].reshape(NW, NCH, K)
    srce = edge_index[0].reshape(NS, KCH, K)
    dste = edge_index[1].reshape(NS, KCH, K)
    data_p = jnp.pad(data, ((0, NP - N), (0, 0)))
    degp = _deg_sc(dstd)
    table1, self1 = _tc_pre(data_p, W1, degp)
    p1 = _edge_sc(table1, srce, dste)
    x1, table2, self2 = _tc_mid(p1, degp, self1, b1.reshape(1, D), W2)
    p2 = _edge_sc(table2, srce, dste)
    x2 = _tc_post(p2, degp, self2, b2.reshape(1, D))
    return (x2[:N], x1[:N], x2[:N])


# speedup vs baseline: 25.5651x; 1.3707x over previous
import functools

import jax
import jax.numpy as jnp
from jax import lax
from jax.experimental import pallas as pl
from jax.experimental.pallas import tpu as pltpu
from jax.experimental.pallas import tpu_sc as plsc

N = 10000
E = 320000
D = 128
HD = D // 2
NP = 10240

NC = 2
NS = 16
NW = NC * NS
K = 80
EPW = E // NW
NCH = EPW // K
EPT = E // NS
KCH = EPT // K
R = NP // NS
ZR = 128

_mesh = plsc.VectorSubcoreMesh(core_axis_name="c", subcore_axis_name="s")


@functools.partial(
    pl.kernel,
    out_type=jax.ShapeDtypeStruct((NC, NP, 16), jnp.float32),
    mesh=_mesh,
    scratch_types=[
        pltpu.VMEM((NCH, K), jnp.int32),
        pltpu.VMEM((K, 16), jnp.float32),
        pltpu.VMEM((R, 16), jnp.float32),
        pltpu.VMEM_SHARED((NP, 16), jnp.float32),
    ],
    compiler_params=pltpu.CompilerParams(use_tc_tiling_on_sc=False),
)
def _deg_sc(dst_hbm, out_hbm, idx_v, ones_v, zero_v, acc_sh):
    c = lax.axis_index("c")
    s = lax.axis_index("s")
    wid = c * NS + s
    pltpu.sync_copy(dst_hbm.at[wid], idx_v)

    def _fill(i, _):
        ones_v[i] = jnp.ones((16,), jnp.float32)
        return 0

    lax.fori_loop(0, K, _fill, 0)

    def _fillz(i, _):
        zero_v[i] = jnp.zeros((16,), jnp.float32)
        return 0

    lax.fori_loop(0, R, _fillz, 0)
    pltpu.sync_copy(zero_v, acc_sh.at[pl.ds(s * R, R)])
    plsc.subcore_barrier()

    def _body(j, _):
        pltpu.sync_copy(ones_v, acc_sh.at[idx_v.at[j]], add=True)
        return 0

    lax.fori_loop(0, NCH, _body, 0)
    plsc.subcore_barrier()
    pltpu.sync_copy(acc_sh.at[pl.ds(s * R, R)], out_hbm.at[c].at[pl.ds(s * R, R)])


@functools.partial(
    pl.kernel,
    out_type=jax.ShapeDtypeStruct((NC, NP, HD), jnp.float32),
    mesh=_mesh,
    scratch_types=[
        pltpu.VMEM((KCH, K), jnp.int32),
        pltpu.VMEM((KCH, K), jnp.int32),
        [pltpu.VMEM((K, HD), jnp.float32)] * 5,
        pltpu.VMEM((ZR, HD), jnp.float32),
        pltpu.VMEM_SHARED((NP, HD), jnp.float32),
        [pltpu.SemaphoreType.DMA] * 5,
        pltpu.SemaphoreType.DMA,
        pltpu.SemaphoreType.DMA,
    ],
    compiler_params=pltpu.CompilerParams(use_tc_tiling_on_sc=False),
)
def _edge_sc(table_hbm, src_hbm, dst_hbm, out_hbm,
             src_v, dst_v, rows, zero_v, acc_sh,
             gsem, isem0, isem1):
    c = lax.axis_index("c")
    s = lax.axis_index("s")
    tb = table_hbm.at[c]
    pltpu.async_copy(src_hbm.at[s], src_v, isem0)
    pltpu.async_copy(dst_hbm.at[s], dst_v, isem1)

    def _fillz(i, _):
        for q in range(HD // 16):
            zero_v[i, pl.ds(q * 16, 16)] = jnp.zeros((16,), jnp.float32)
        return 0

    lax.fori_loop(0, ZR, _fillz, 0)
    for t in range(R // ZR):
        pltpu.sync_copy(zero_v, acc_sh.at[pl.ds(s * R + t * ZR, ZR)])
    pltpu.make_async_copy(src_hbm.at[s], src_v, isem0).wait()
    pltpu.make_async_copy(dst_hbm.at[s], dst_v, isem1).wait()
    plsc.subcore_barrier()

    for p in range(5):
        pltpu.async_copy(tb.at[src_v.at[p]], rows[p], gsem[p])

    def _body(jj, _):
        j0 = 5 * jj
        for p in range(5):
            j = j0 + p
            pltpu.make_async_copy(tb.at[src_v.at[j]], rows[p], gsem[p]).wait()
            pltpu.sync_copy(rows[p], acc_sh.at[dst_v.at[j]], add=True)

            @pl.when(j + 5 < KCH)
            def _():
                pltpu.async_copy(tb.at[src_v.at[j + 5]], rows[p], gsem[p])

        return 0

    lax.fori_loop(0, KCH // 5, _body, 0)
    plsc.subcore_barrier()
    pltpu.sync_copy(acc_sh.at[pl.ds(s * R, R)], out_hbm.at[c].at[pl.ds(s * R, R)])


_BR = 1024


def _norm_block(degp):
    deg = degp[0, :, 0:1] + degp[1, :, 0:1] + 1.0
    return lax.rsqrt(deg)


def _split_cols(m):
    return jnp.stack([m[:, :HD], m[:, HD:]])


def _tc_pre_body(x_ref, w_ref, degp_ref, table_ref, self_ref):
    norm = _norm_block(degp_ref[...])
    h = jnp.dot(x_ref[...], w_ref[...], preferred_element_type=jnp.float32)
    table_ref[...] = _split_cols(h * norm)
    self_ref[...] = h * (norm * norm)


def _tc_mid_body(p_ref, degp_ref, self_ref, b_ref, w_ref,
                 x_ref, table_ref, self2_ref):
    norm = _norm_block(degp_ref[...])
    p = jnp.concatenate([p_ref[0], p_ref[1]], axis=1)
    agg = p * norm + self_ref[...] + b_ref[...]
    x1 = jnp.maximum(agg, 0.0)
    x_ref[...] = x1
    h2 = jnp.dot(x1, w_ref[...], preferred_element_type=jnp.float32)
    table_ref[...] = _split_cols(h2 * norm)
    self2_ref[...] = h2 * (norm * norm)


def _tc_post_body(p_ref, degp_ref, self_ref, b_ref, x_ref):
    norm = _norm_block(degp_ref[...])
    p = jnp.concatenate([p_ref[0], p_ref[1]], axis=1)
    agg = p * norm + self_ref[...] + b_ref[...]
    x_ref[...] = jnp.maximum(agg, 0.0)


_row_spec = pl.BlockSpec((_BR, D), lambda i: (i, 0))
_t_spec = pl.BlockSpec((NC, _BR, HD), lambda i: (0, i, 0))
_degp_spec = pl.BlockSpec((NC, _BR, 16), lambda i: (0, i, 0))
_w_spec = pl.BlockSpec((D, D), lambda i: (0, 0))
_b_spec = pl.BlockSpec((1, D), lambda i: (0, 0))
_grid = (NP // _BR,)

_tc_pre = pl.pallas_call(
    _tc_pre_body,
    grid=_grid,
    in_specs=[_row_spec, _w_spec, _degp_spec],
    out_specs=[_t_spec, _row_spec],
    out_shape=[jax.ShapeDtypeStruct((NC, NP, HD), jnp.float32),
               jax.ShapeDtypeStruct((NP, D), jnp.float32)],
)

_tc_mid = pl.pallas_call(
    _tc_mid_body,
    grid=_grid,
    in_specs=[_t_spec, _degp_spec, _row_spec, _b_spec, _w_spec],
    out_specs=[_row_spec, _t_spec, _row_spec],
    out_shape=[jax.ShapeDtypeStruct((NP, D), jnp.float32),
               jax.ShapeDtypeStruct((NC, NP, HD), jnp.float32),
               jax.ShapeDtypeStruct((NP, D), jnp.float32)],
)

_tc_post = pl.pallas_call(
    _tc_post_body,
    grid=_grid,
    in_specs=[_t_spec, _degp_spec, _row_spec, _b_spec],
    out_specs=_row_spec,
    out_shape=jax.ShapeDtypeStruct((NP, D), jnp.float32),
)


def kernel(data, edge_index, W1, b1, W2, b2):
    srcd = edge_index[0].reshape(NW, NCH, K)
    dstd = edge_index[1].reshape(NW, NCH, K)
    srce = edge_index[0].reshape(NS, KCH, K)
    dste = edge_index[1].reshape(NS, KCH, K)
    data_p = jnp.pad(data, ((0, NP - N), (0, 0)))
    degp = _deg_sc(dstd)
    table1, self1 = _tc_pre(data_p, W1, degp)
    p1 = _edge_sc(table1, srce, dste)
    x1, table2, self2 = _tc_mid(p1, degp, self1, b1.reshape(1, D), W2)
    p2 = _edge_sc(table2, srce, dste)
    x2 = _tc_post(p2, degp, self2, b2.reshape(1, D))
    return (x2[:N], x1[:N], x2[:N])
